# baseline (speedup 1.0000x reference)
import functools

import jax
import jax.numpy as jnp
from jax import lax
from jax.experimental import pallas as pl
from jax.experimental.pallas import tpu as pltpu
from jax.experimental.pallas import tpu_sc as plsc

N_SAMP = 20
LANES = 16


def _rsqrt_nr(x):
    i = lax.bitcast_convert_type(x, jnp.int32)
    i = jnp.int32(0x5F3759DF) - lax.shift_right_arithmetic(i, jnp.int32(1))
    y = lax.bitcast_convert_type(i, jnp.float32)
    for _ in range(3):
        y = y * (jnp.float32(1.5) - jnp.float32(0.5) * x * y * y)
    return y


@functools.cache
def _build_sc(B, V, D):
    NC, NS = 2, 16
    NW = NC * NS
    PER_W = B // NW
    CB = 64
    NCHUNK = PER_W // CB
    K = 2 + N_SAMP
    ROWS = CB * K
    NGRP = ROWS // LANES
    NZROWS = CB * N_SAMP // 128

    mesh = plsc.VectorSubcoreMesh(
        core_axis_name="c", subcore_axis_name="s", num_cores=NC, num_subcores=NS)

    @functools.partial(
        pl.kernel,
        out_type=jax.ShapeDtypeStruct((K, D, B), jnp.float32),
        mesh=mesh,
        scratch_types=[
            pltpu.VMEM((2, CB), jnp.int32),
            pltpu.VMEM((2, CB * N_SAMP), jnp.int32),
            pltpu.VMEM((2, ROWS, D // 2), jnp.float32),
            pltpu.VMEM((2, ROWS, D // 2), jnp.float32),
            pltpu.VMEM((K, D, CB), jnp.float32),
            pltpu.VMEM((ROWS,), jnp.int32),
            pltpu.VMEM((LANES,), jnp.float32),
            pltpu.SemaphoreType.DMA,
            pltpu.SemaphoreType.DMA,
        ],
        compiler_params=pltpu.CompilerParams(
            use_tc_tiling_on_sc=False, needs_layout_passes=False),
    )
    def sc(ctxr_h, tgt_h, nz_h, dv_h, tta_h, ttb_h, out_h,
           idx_t, idx_nz, bufa, bufb, slab, srcmap, dv_v, sem0, sem1):
        wid = lax.axis_index("s") * NC + lax.axis_index("c")
        sems = (sem0, sem1)
        pltpu.sync_copy(dv_h, dv_v)

        def mk(g, _):
            p = g * LANES + lax.iota(jnp.int32, LANES)
            bl = lax.div(p, jnp.int32(K))
            k = p - bl * K
            src = jnp.where(
                k == 0, bl,
                jnp.where(k == 1, CB + bl, 2 * CB + bl * N_SAMP + (k - 2)))
            srcmap[pl.ds(g * LANES, LANES)] = src
            return 0
        lax.fori_loop(0, NGRP, mk, 0)
        dvec = dv_v[...]

        def load(ci, slot):
            gb = pl.multiple_of(wid * PER_W + ci * CB, CB)
            pltpu.sync_copy(tgt_h.at[pl.ds(gb, CB)], idx_t.at[slot])
            nzoff = pl.multiple_of(gb * N_SAMP, CB * N_SAMP)
            pltpu.sync_copy(nz_h.at[pl.ds(nzoff, CB * N_SAMP)], idx_nz.at[slot])
            sem = sems[slot]
            cps = []
            for th, bf in ((tta_h, bufa), (ttb_h, bufb)):
                half = 0 if bf is bufa else 1
                cps.append(pltpu.async_copy(
                    ctxr_h.at[pl.ds(gb, CB), pl.ds(half * (D // 2), D // 2)],
                    bf.at[slot, pl.ds(0, CB)], sem))
                cps.append(pltpu.async_copy(
                    th.at[idx_t.at[slot]], bf.at[slot, pl.ds(CB, CB)], sem))
                for j in range(NZROWS):
                    cps.append(pltpu.async_copy(
                        th.at[idx_nz.at[slot, pl.ds(j * 128, 128)]],
                        bf.at[slot, pl.ds(2 * CB + j * 128, 128)], sem))
            return cps

        def compute(ci, slot):
            gb = pl.multiple_of(wid * PER_W + ci * CB, CB)
            bufas, bufbs = bufa.at[slot], bufb.at[slot]

            def grp(g, _):
                base = g * LANES
                p = base + lax.iota(jnp.int32, LANES)
                bl = lax.div(p, jnp.int32(K))
                k = p - bl * K
                src = srcmap[pl.ds(base, LANES)]
                cols = [plsc.load_gather(
                            bufas if j < D // 2 else bufbs,
                            [src, jnp.full((LANES,), j % (D // 2), jnp.int32)])
                        for j in range(D)]
                ss = cols[0] * cols[0]
                for j in range(1, D):
                    ss = ss + cols[j] * cols[j]
                scale = jnp.where(ss > 1.0, _rsqrt_nr(ss), jnp.float32(1.0))
                for j in range(D):
                    plsc.store_scatter(
                        slab, [k, jnp.full((LANES,), j, jnp.int32), bl],
                        cols[j] * scale + dvec)
                return 0
            lax.fori_loop(0, NGRP, grp, 0)
            pltpu.sync_copy(slab, out_h.at[:, :, pl.ds(gb, CB)])

        pend = load(0, 0)
        for ci in range(NCHUNK):
            slot = ci % 2
            if ci + 1 < NCHUNK:
                nxt = load(ci + 1, 1 - slot)
            else:
                nxt = None
            for cp in pend:
                cp.wait()
            compute(ci, slot)
            pend = nxt

    return sc


def kernel(contexts, target, n_samples, context_table, target_table):
    B = contexts.shape[0]
    V, D = target_table.shape
    noise = jax.random.randint(
        jax.random.key(42), (B * N_SAMP,), 0, V, dtype=jnp.int32)
    delta = (jnp.asarray(n_samples) - N_SAMP).astype(jnp.float32)
    dvec = jnp.full((LANES,), 1.0, jnp.float32) * delta
    ctx_rows = jnp.take(context_table, contexts, axis=0)
    tta = target_table[:, : D // 2]
    ttb = target_table[:, D // 2:]
    out = _build_sc(B, V, D)(
        ctx_rows, target, noise, dvec, tta, ttb)
    return out.transpose(2, 0, 1)


# speedup vs baseline: 2.2075x; 2.2075x over previous
import functools

import jax
import jax.numpy as jnp
from jax import lax
from jax.experimental import pallas as pl
from jax.experimental.pallas import tpu as pltpu
from jax.experimental.pallas import tpu_sc as plsc

N_SAMP = 20
LANES = 16


def _rsqrt_nr(x):
    i = lax.bitcast_convert_type(x, jnp.int32)
    i = jnp.int32(0x5F3759DF) - lax.shift_right_arithmetic(i, jnp.int32(1))
    y = lax.bitcast_convert_type(i, jnp.float32)
    for _ in range(3):
        y = y * (jnp.float32(1.5) - jnp.float32(0.5) * x * y * y)
    return y


@functools.cache
def _build_sc(B, V, D):
    NC, NS = 2, 16
    NW = NC * NS
    PER_W = B // NW
    CB = 64
    NCHUNK = PER_W // CB
    K = 2 + N_SAMP
    ROWS = CB * K
    NGRP = ROWS // LANES
    NZROWS = CB * N_SAMP // 128

    mesh = plsc.VectorSubcoreMesh(
        core_axis_name="c", subcore_axis_name="s", num_cores=NC, num_subcores=NS)

    @functools.partial(
        pl.kernel,
        out_type=jax.ShapeDtypeStruct((K, D, B), jnp.float32),
        mesh=mesh,
        scratch_types=[
            pltpu.VMEM((PER_W,), jnp.int32),
            pltpu.VMEM((PER_W * N_SAMP,), jnp.int32),
            pltpu.VMEM((2, ROWS, D), jnp.float32),
            pltpu.VMEM((2, K, D, CB), jnp.float32),
            pltpu.VMEM((ROWS,), jnp.int32),
            pltpu.VMEM((LANES,), jnp.float32),
            pltpu.SemaphoreType.DMA,
            pltpu.SemaphoreType.DMA,
            pltpu.SemaphoreType.DMA,
            pltpu.SemaphoreType.DMA,
        ],
        compiler_params=pltpu.CompilerParams(
            use_tc_tiling_on_sc=False, needs_layout_passes=False),
    )
    def sc(ctxr_h, tgt_h, nz_h, dv_h, tt_h, out_h,
           idx_t, idx_nz, buf, slab, srcmap, dv_v, sem0, sem1, semo0, semo1):
        wid = lax.axis_index("s") * NC + lax.axis_index("c")
        sems = (sem0, sem1)
        osems = (semo0, semo1)
        pltpu.sync_copy(dv_h, dv_v)
        wb = pl.multiple_of(wid * PER_W, PER_W)
        pltpu.sync_copy(tgt_h.at[pl.ds(wb, PER_W)], idx_t)
        wn = pl.multiple_of(wid * PER_W * N_SAMP, PER_W * N_SAMP)
        pltpu.sync_copy(nz_h.at[pl.ds(wn, PER_W * N_SAMP)], idx_nz)

        def mk(g, _):
            p = g * LANES + lax.iota(jnp.int32, LANES)
            bl = lax.div(p, jnp.int32(K))
            k = p - bl * K
            src = jnp.where(
                k == 0, bl,
                jnp.where(k == 1, CB + bl, 2 * CB + bl * N_SAMP + (k - 2)))
            srcmap[pl.ds(g * LANES, LANES)] = src
            return 0
        lax.fori_loop(0, NGRP, mk, 0)
        dvec = dv_v[...]

        def load(ci, slot):
            gb = pl.multiple_of(wid * PER_W + ci * CB, CB)
            sem = sems[slot]
            cps = [
                pltpu.async_copy(ctxr_h.at[pl.ds(gb, CB)],
                                 buf.at[slot, pl.ds(0, CB)], sem),
                pltpu.async_copy(tt_h.at[idx_t.at[pl.ds(ci * CB, CB)]],
                                 buf.at[slot, pl.ds(CB, CB)], sem),
            ]
            for j in range(NZROWS):
                cps.append(pltpu.async_copy(
                    tt_h.at[idx_nz.at[pl.ds(ci * CB * N_SAMP + j * 128, 128)]],
                    buf.at[slot, pl.ds(2 * CB + j * 128, 128)], sem))
            return cps

        def compute(ci, slot):
            gb = pl.multiple_of(wid * PER_W + ci * CB, CB)
            bufs = buf.at[slot]
            slabs = slab.at[slot]

            def grp(g, _):
                base = g * LANES
                p = base + lax.iota(jnp.int32, LANES)
                bl = lax.div(p, jnp.int32(K))
                k = p - bl * K
                src = srcmap[pl.ds(base, LANES)]
                cols = [plsc.load_gather(bufs, [src, jnp.full((LANES,), j, jnp.int32)])
                        for j in range(D)]
                ss = cols[0] * cols[0]
                for j in range(1, D):
                    ss = ss + cols[j] * cols[j]
                scale = jnp.where(ss > 1.0, _rsqrt_nr(ss), jnp.float32(1.0))
                for j in range(D):
                    plsc.store_scatter(
                        slabs, [k, jnp.full((LANES,), j, jnp.int32), bl],
                        cols[j] * scale + dvec)
                return 0
            lax.fori_loop(0, NGRP, grp, 0)
            return pltpu.async_copy(slabs, out_h.at[:, :, pl.ds(gb, CB)], osems[slot])

        pend = load(0, 0)
        owrites = [None, None]
        for ci in range(NCHUNK):
            slot = ci % 2
            if ci + 1 < NCHUNK:
                nxt = load(ci + 1, 1 - slot)
            else:
                nxt = None
            for cp in pend:
                cp.wait()
            if owrites[slot] is not None:
                owrites[slot].wait()
            owrites[slot] = compute(ci, slot)
            pend = nxt
        for ow in owrites:
            if ow is not None:
                ow.wait()

    return sc


def kernel(contexts, target, n_samples, context_table, target_table):
    B = contexts.shape[0]
    V, D = target_table.shape
    noise = jax.random.randint(
        jax.random.key(42), (B * N_SAMP,), 0, V, dtype=jnp.int32)
    delta = (jnp.asarray(n_samples) - N_SAMP).astype(jnp.float32)
    dvec = jnp.full((LANES,), 1.0, jnp.float32) * delta
    ctx_rows = jnp.take(context_table, contexts, axis=0)
    out = _build_sc(B, V, D)(
        ctx_rows, target, noise, dvec, target_table)
    return out.transpose(2, 0, 1)
